# baseline (speedup 1.0000x reference)
import functools

import jax
import jax.numpy as jnp
from jax import lax
from jax.experimental import pallas as pl
from jax.experimental.pallas import tpu as pltpu
from jax.experimental.pallas import tpu_sc as plsc

_NUM_CORES = 2
_NUM_SUBCORES = 16
_NUM_WORKERS = _NUM_CORES * _NUM_SUBCORES


def _sc_gather(table, ids, seq, hidden_dim):
    b_per_w = seq // _NUM_WORKERS
    mesh = plsc.VectorSubcoreMesh(core_axis_name="c", subcore_axis_name="s")

    @functools.partial(
        pl.kernel,
        mesh=mesh,
        out_type=jax.ShapeDtypeStruct((seq, hidden_dim), jnp.float32),
        scratch_types=[
            pltpu.VMEM((b_per_w,), jnp.int32),
            pltpu.VMEM((b_per_w, hidden_dim), jnp.float32),
            pltpu.SemaphoreType.DMA,
        ],
        compiler_params=pltpu.CompilerParams(use_tc_tiling_on_sc=True),
    )
    def gather_kernel(table_hbm, idx_hbm, out_hbm, idx_v, rows_v, sem):
        wid = lax.axis_index("s") * _NUM_CORES + lax.axis_index("c")
        base = wid * b_per_w
        pltpu.sync_copy(idx_hbm.at[pl.ds(base, b_per_w)], idx_v)
        pltpu.async_copy(table_hbm.at[idx_v], rows_v, sem).wait()
        pltpu.sync_copy(rows_v, out_hbm.at[pl.ds(base, b_per_w)])

    return gather_kernel(table, ids)


def _tc_logits_t(hidden, table, block_v=3400):
    seq, hidden_dim = hidden.shape
    vocab = table.shape[0]

    def mm_body(w_ref, h_ref, o_ref):
        o_ref[...] = lax.dot_general(
            w_ref[...].astype(jnp.bfloat16),
            h_ref[...].astype(jnp.bfloat16),
            (((1,), (1,)), ((), ())),
            preferred_element_type=jnp.float32,
        )

    return pl.pallas_call(
        mm_body,
        grid=(pl.cdiv(vocab, block_v),),
        in_specs=[
            pl.BlockSpec((block_v, hidden_dim), lambda i: (i, 0)),
            pl.BlockSpec((seq, hidden_dim), lambda i: (0, 0)),
        ],
        out_specs=pl.BlockSpec((block_v, seq), lambda i: (i, 0)),
        out_shape=jax.ShapeDtypeStruct((vocab, seq), jnp.float32),
        compiler_params=pltpu.CompilerParams(
            vmem_limit_bytes=120 * 1024 * 1024,
        ),
    )(table, hidden)


def kernel(input_ids, embed_weight):
    batch, seq = input_ids.shape
    vocab, hidden_dim = embed_weight.shape
    ids = input_ids.reshape(batch * seq).astype(jnp.int32)
    hidden = _sc_gather(embed_weight, ids, batch * seq, hidden_dim)
    logits_t = _tc_logits_t(hidden, embed_weight)
    return logits_t.T.reshape(batch, seq, vocab)


# speedup vs baseline: 1.0008x; 1.0008x over previous
import functools

import jax
import jax.numpy as jnp
from jax import lax
from jax.experimental import pallas as pl
from jax.experimental.pallas import tpu as pltpu
from jax.experimental.pallas import tpu_sc as plsc

_NUM_CORES = 2
_NUM_SUBCORES = 16
_NUM_WORKERS = _NUM_CORES * _NUM_SUBCORES


def _sc_gather(table, ids, seq, hidden_dim):
    b_per_w = seq // _NUM_WORKERS
    mesh = plsc.VectorSubcoreMesh(core_axis_name="c", subcore_axis_name="s")

    @functools.partial(
        pl.kernel,
        mesh=mesh,
        out_type=jax.ShapeDtypeStruct((seq, hidden_dim), jnp.float32),
        scratch_types=[
            pltpu.VMEM((b_per_w,), jnp.int32),
            pltpu.VMEM((b_per_w, hidden_dim), jnp.float32),
            pltpu.SemaphoreType.DMA,
        ],
        compiler_params=pltpu.CompilerParams(use_tc_tiling_on_sc=True),
    )
    def gather_kernel(table_hbm, idx_hbm, out_hbm, idx_v, rows_v, sem):
        wid = lax.axis_index("s") * _NUM_CORES + lax.axis_index("c")
        base = wid * b_per_w
        pltpu.sync_copy(idx_hbm.at[pl.ds(base, b_per_w)], idx_v)
        pltpu.async_copy(table_hbm.at[idx_v], rows_v, sem).wait()
        pltpu.sync_copy(rows_v, out_hbm.at[pl.ds(base, b_per_w)])

    return gather_kernel(table, ids)


def _tc_logits_t(hidden, table, block_v=3200):
    seq, hidden_dim = hidden.shape
    vocab = table.shape[0]

    def mm_body(w_ref, h_ref, o_ref):
        o_ref[...] = lax.dot_general(
            w_ref[...].astype(jnp.bfloat16),
            h_ref[...].astype(jnp.bfloat16),
            (((1,), (1,)), ((), ())),
            preferred_element_type=jnp.float32,
        )

    return pl.pallas_call(
        mm_body,
        grid=(pl.cdiv(vocab, block_v),),
        in_specs=[
            pl.BlockSpec((block_v, hidden_dim), lambda i: (i, 0)),
            pl.BlockSpec((seq, hidden_dim), lambda i: (0, 0)),
        ],
        out_specs=pl.BlockSpec((block_v, seq), lambda i: (i, 0)),
        out_shape=jax.ShapeDtypeStruct((vocab, seq), jnp.float32),
        compiler_params=pltpu.CompilerParams(
            vmem_limit_bytes=120 * 1024 * 1024,
        ),
    )(table, hidden)


def kernel(input_ids, embed_weight):
    batch, seq = input_ids.shape
    vocab, hidden_dim = embed_weight.shape
    ids = input_ids.reshape(batch * seq).astype(jnp.int32)
    hidden = _sc_gather(embed_weight, ids, batch * seq, hidden_dim)
    logits_t = _tc_logits_t(hidden, embed_weight)
    return logits_t.T.reshape(batch, seq, vocab)
